# baseline (speedup 1.0000x reference)
import functools

import jax
import jax.numpy as jnp
from jax import lax
from jax.experimental import pallas as pl
from jax.experimental.pallas import tpu as pltpu
from jax.experimental.pallas import tpu_sc as plsc

N_H, N_C, N_O = 40000, 40000, 20000
NS_LIST = (N_H, N_C, N_O)
E = 360000
NCORE, NSUB, NTILE = 2, 16, 32
CHUNK = 128
NCH = 88
E_PAD = NTILE * NCH * CHUNK
KBUF = 11
F = 16
HIGH = None
f32 = jnp.float32


def _i32c(v):
    return jnp.int32(v) if isinstance(v, int) else v


def _im(f):
    return lambda *a: tuple(_i32c(v) for v in f(*a))


def _dot_t(a, w):
    return lax.dot_general(a, w, (((1,), (1,)), ((), ())),
                           preferred_element_type=f32, precision=HIGH)


def _dot(a, b):
    return lax.dot_general(a, b, (((1,), (0,)), ((), ())),
                           preferred_element_type=f32, precision=HIGH)


def _enc_body(x_ref, w1_ref, b1_ref, w2_ref, b2_ref, wg_ref,
              h_ref, g0_ref, g1_ref, g2_ref):
    z = jnp.maximum(_dot_t(x_ref[...], w1_ref[...]) + b1_ref[...], 0.0)
    h = jnp.maximum(_dot_t(z, w2_ref[...]) + b2_ref[...], 0.0)
    h_ref[...] = h
    g = _dot(h, wg_ref[...])
    g0_ref[...] = g[:, 0:16]
    g1_ref[...] = g[:, 16:32]
    g2_ref[...] = g[:, 32:48]


def _encode(x, w1, b1, w2, b2, wg, blk):
    n = x.shape[0]
    nb = n // blk
    full = lambda *shape: pl.BlockSpec(shape, _im(lambda i: tuple(0 for _ in shape)))
    return pl.pallas_call(
        _enc_body,
        grid=(nb,),
        in_specs=[
            pl.BlockSpec((blk, 128), _im(lambda i: (i, 0))),
            full(32, 128), full(1, 32), full(16, 32), full(1, 16),
            full(16, 48),
        ],
        out_specs=[pl.BlockSpec((blk, F), _im(lambda i: (i, 0)))] * 4,
        out_shape=[jax.ShapeDtypeStruct((n, F), f32)] * 4,
    )(x, w1, b1, w2, b2, wg)


def _comb_body(p0_ref, p1_ref, h_ref, wr_ref, bs_ref, wg_ref,
               h1_ref, g0_ref, g1_ref):
    v = p0_ref[0] + p1_ref[0] + _dot_t(h_ref[...], wr_ref[...]) + bs_ref[...]
    h1 = jnp.maximum(v, 0.0)
    h1_ref[...] = h1
    g = _dot(h1, wg_ref[...])
    g0_ref[...] = g[:, 0:16]
    g1_ref[...] = g[:, 16:32]


def _combine(p, h, wr, bs, wg, blk):
    n = h.shape[0]
    nb = n // blk
    full = lambda *shape: pl.BlockSpec(shape, _im(lambda i: tuple(0 for _ in shape)))
    return pl.pallas_call(
        _comb_body,
        grid=(nb,),
        in_specs=[
            pl.BlockSpec((1, blk, F), _im(lambda i: (0, i, 0))),
            pl.BlockSpec((1, blk, F), _im(lambda i: (1, i, 0))),
            pl.BlockSpec((blk, F), _im(lambda i: (i, 0))),
            full(16, 16), full(1, 16), full(16, 32),
        ],
        out_specs=[pl.BlockSpec((blk, F), _im(lambda i: (i, 0)))] * 3,
        out_shape=[jax.ShapeDtypeStruct((n, F), f32)] * 3,
    )(p, p, h, wr, bs, wg)


def _final_body(p0_ref, p1_ref, h_ref, wr_ref, bs_ref, wp_ref, bp_ref, o_ref):
    v = p0_ref[0] + p1_ref[0] + _dot_t(h_ref[...], wr_ref[...]) + bs_ref[...]
    h1 = jnp.maximum(v, 0.0)
    o_ref[...] = jnp.sum(h1 * wp_ref[...], axis=1, keepdims=True) + bp_ref[...]


def _final(p, h, wr, bs, wp, bp, blk):
    n = h.shape[0]
    nb = n // blk
    full = lambda *shape: pl.BlockSpec(shape, _im(lambda i: tuple(0 for _ in shape)))
    return pl.pallas_call(
        _final_body,
        grid=(nb,),
        in_specs=[
            pl.BlockSpec((1, blk, F), _im(lambda i: (0, i, 0))),
            pl.BlockSpec((1, blk, F), _im(lambda i: (1, i, 0))),
            pl.BlockSpec((blk, F), _im(lambda i: (i, 0))),
            full(16, 16), full(1, 16), full(1, 16), full(1, 1),
        ],
        out_specs=pl.BlockSpec((blk, 1), _im(lambda i: (i, 0))),
        out_shape=jax.ShapeDtypeStruct((n, 1), f32),
    )(p, p, h, wr, bs, wp, bp)


def _make_sc_scatter(rel_dst, nd_list):
    nrel = len(rel_dst)
    ndst = len(nd_list)
    npad_list = [nd + 32 for nd in nd_list]

    scratch = [pltpu.VMEM_SHARED((npad, F), f32) for npad in npad_list]
    scratch += [
        pltpu.VMEM((NCH // 4, CHUNK), jnp.int32),
        pltpu.VMEM((NCH // 4, CHUNK), jnp.int32),
    ]
    scratch += [pltpu.VMEM((CHUNK, F), f32) for _ in range(KBUF)]
    scratch += [pltpu.VMEM((CHUNK, F), f32)]
    scratch += [pltpu.SemaphoreType.DMA for _ in range(KBUF)]
    out_type = [jax.ShapeDtypeStruct((NCORE, nd, F), f32) for nd in nd_list]
    mesh = plsc.VectorSubcoreMesh(core_axis_name="c", subcore_axis_name="s")

    @functools.partial(
        pl.kernel, out_type=out_type, mesh=mesh, scratch_types=scratch,
        compiler_params=pltpu.CompilerParams(use_tc_tiling_on_sc=False))
    def sc_kernel(*refs):
        ins = refs[:3 * nrel]
        outs = refs[3 * nrel:3 * nrel + ndst]
        accs = refs[3 * nrel + ndst:3 * nrel + 2 * ndst]
        rest = refs[3 * nrel + 2 * ndst:]
        srcbuf, dstbuf = rest[0], rest[1]
        rows = rest[2:2 + KBUF]
        zbuf = rest[2 + KBUF]
        sems = rest[3 + KBUF:3 + 2 * KBUF]
        rows0 = rows[0]

        cid = lax.axis_index("c")
        sid = lax.axis_index("s")
        wid = sid * NCORE + cid

        i32 = jnp.int32

        @pl.loop(i32(0), i32(CHUNK))
        def zstore(i):
            zbuf[i, :] = jnp.zeros((F,), f32)

        for acc, npad in zip(accs, npad_list):
            nfull, rem = divmod(npad, CHUNK)

            @pl.loop(sid, i32(nfull), step=i32(NSUB))
            def zcopy(c, acc=acc):
                pltpu.sync_copy(zbuf, acc.at[pl.ds(c * CHUNK, CHUNK)])
            if rem:
                @pl.when(sid == i32(nfull % NSUB))
                def _(acc=acc, off=nfull * CHUNK, rem=rem):
                    pltpu.sync_copy(zbuf.at[pl.ds(i32(0), rem)],
                                    acc.at[pl.ds(i32(off), rem)])

        plsc.subcore_barrier()

        for r in range(nrel):
            tab = ins[3 * r]
            src = ins[3 * r + 1]
            dst = ins[3 * r + 2]
            acc = accs[rel_dst[r]]

            half = NCH // 4
            for h in range(4):
                pltpu.sync_copy(
                    src.at[pl.ds(wid * NCH + h * half, half)], srcbuf)
                pltpu.sync_copy(
                    dst.at[pl.ds(wid * NCH + h * half, half)], dstbuf)

                for b in range(KBUF):
                    pltpu.async_copy(tab.at[srcbuf.at[i32(b)]],
                                     rows[b], sems[b])

                nfull = half // KBUF
                tail = half - nfull * KBUF

                @pl.loop(i32(0), i32(nfull))
                def step(i, tab=tab, acc=acc):
                    j0 = KBUF * i
                    for b in range(KBUF):
                        j = j0 + i32(b)
                        pltpu.make_async_copy(
                            tab.at[srcbuf.at[i32(b)]], rows[b], sems[b]).wait()
                        pltpu.sync_copy(rows[b], acc.at[dstbuf.at[j]],
                                        add=True)
                        pltpu.async_copy(
                            tab.at[srcbuf.at[jnp.minimum(j + KBUF, half - 1)]],
                            rows[b], sems[b])
                for b in range(KBUF):
                    pltpu.make_async_copy(
                        tab.at[srcbuf.at[i32(b)]], rows[b], sems[b]).wait()
                    if b < tail:
                        pltpu.sync_copy(
                            rows[b], acc.at[dstbuf.at[i32(nfull * KBUF + b)]],
                            add=True)

        plsc.subcore_barrier()

        for out, acc, nd in zip(outs, accs, nd_list):
            nfull, rem = divmod(nd, CHUNK)

            @pl.loop(sid, i32(nfull), step=i32(NSUB))
            def wcopy(c, out=out, acc=acc):
                off = pl.multiple_of(c * CHUNK, CHUNK)
                pltpu.sync_copy(acc.at[pl.ds(off, CHUNK)], rows0)
                pltpu.sync_copy(rows0, out.at[cid, pl.ds(off, CHUNK)])
            if rem:
                @pl.when(sid == i32(nfull % NSUB))
                def _(out=out, acc=acc, off=nfull * CHUNK, rem=rem):
                    pltpu.sync_copy(acc.at[pl.ds(i32(off), rem)],
                                    rows0.at[pl.ds(i32(0), rem)])
                    pltpu.sync_copy(rows0.at[pl.ds(i32(0), rem)],
                                    out.at[cid, pl.ds(i32(off), rem)])

    return sc_kernel


def kernel(x_H, x_C, x_Others, encW1, encb1, encW2, encb2, Wrel, brel, Wroot,
           WpH, bpH, WpC, bpC, edge_index_H_H, edge_index_H_C,
           edge_index_H_Others, edge_index_C_H, edge_index_C_C,
           edge_index_C_Others, edge_index_Others_H, edge_index_Others_C,
           edge_index_Others_Others):
    eis = [edge_index_H_H, edge_index_H_C, edge_index_H_Others,
           edge_index_C_H, edge_index_C_C, edge_index_C_Others,
           edge_index_Others_H, edge_index_Others_C, edge_index_Others_Others]
    xs = [x_H, x_C, x_Others]

    pad = E_PAD - E
    srcs, dsts = [], []
    for r in range(9):
        nd = NS_LIST[r % 3]
        ei = eis[r].astype(jnp.int32)
        src = jnp.concatenate([ei[0], jnp.zeros((pad,), jnp.int32)])
        dst = jnp.concatenate([ei[1], jnp.full((pad,), nd, jnp.int32)])
        srcs.append(src.reshape(NTILE * NCH, CHUNK))
        dsts.append(dst.reshape(NTILE * NCH, CHUNK))

    b1 = encb1.reshape(3, 1, 32)
    b2 = encb2.reshape(3, 1, 16)
    wroot_sum = [[Wroot[l, d] + Wroot[l, 3 + d] + Wroot[l, 6 + d]
                  for d in range(3)] for l in range(2)]
    bsum = [[(brel[l, d] + brel[l, 3 + d] + brel[l, 6 + d]).reshape(1, 16)
             for d in range(3)] for l in range(2)]

    h0, g0 = [], []
    for t in range(3):
        wgcat = jnp.concatenate([Wrel[0, 3 * t + k].T for k in range(3)],
                                axis=1)
        res = _encode(xs[t], encW1[t], b1[t], encW2[t], b2[t], wgcat, 4000)
        h0.append(res[0])
        g0.append(res[1:])

    sc0 = _make_sc_scatter(tuple(r % 3 for r in range(9)), NS_LIST)
    args0 = []
    for r in range(9):
        args0 += [g0[r // 3][r % 3], srcs[r], dsts[r]]
    p0 = sc0(*args0)

    h1, g1 = [], []
    for t in range(3):
        wgcat = jnp.concatenate([Wrel[1, 3 * t + k].T for k in range(2)],
                                axis=1)
        res = _combine(p0[t], h0[t], wroot_sum[0][t], bsum[0][t], wgcat, 4000)
        h1.append(res[0])
        g1.append(res[1:])

    rels1 = [3 * s + d for s in range(3) for d in range(2)]
    sc1 = _make_sc_scatter(tuple(r % 3 for r in rels1), (N_H, N_C))
    args1 = []
    for r in rels1:
        args1 += [g1[r // 3][r % 3], srcs[r], dsts[r]]
    p1 = sc1(*args1)

    out_H = _final(p1[0], h1[0], wroot_sum[1][0], bsum[1][0],
                   WpH.reshape(1, 16), bpH.reshape(1, 1), 4000)
    out_C = _final(p1[1], h1[1], wroot_sum[1][1], bsum[1][1],
                   WpC.reshape(1, 16), bpC.reshape(1, 1), 4000)
    return jnp.concatenate([out_H, out_C], axis=0)


# speedup vs baseline: 1.1647x; 1.1647x over previous
import functools

import jax
import jax.numpy as jnp
from jax import lax
from jax.experimental import pallas as pl
from jax.experimental.pallas import tpu as pltpu
from jax.experimental.pallas import tpu_sc as plsc

N_H, N_C, N_O = 40000, 40000, 20000
NS_LIST = (N_H, N_C, N_O)
E = 360000
NCORE, NSUB, NTILE = 2, 16, 32
CHUNK = 128
NCH = 88
E_PAD = NTILE * NCH * CHUNK
KBUF = 8
F = 16
HIGH = None
f32 = jnp.float32


def _i32c(v):
    return jnp.int32(v) if isinstance(v, int) else v


def _im(f):
    return lambda *a: tuple(_i32c(v) for v in f(*a))


def _dot_t(a, w):
    return lax.dot_general(a, w, (((1,), (1,)), ((), ())),
                           preferred_element_type=f32, precision=HIGH)


def _dot(a, b):
    return lax.dot_general(a, b, (((1,), (0,)), ((), ())),
                           preferred_element_type=f32, precision=HIGH)


def _enc_body(x_ref, w1_ref, b1_ref, w2_ref, b2_ref, wg_ref,
              h_ref, g0_ref, g1_ref, g2_ref):
    z = jnp.maximum(_dot_t(x_ref[...], w1_ref[...]) + b1_ref[...], 0.0)
    h = jnp.maximum(_dot_t(z, w2_ref[...]) + b2_ref[...], 0.0)
    h_ref[...] = h
    g = _dot(h, wg_ref[...])
    g0_ref[...] = g[:, 0:16]
    g1_ref[...] = g[:, 16:32]
    g2_ref[...] = g[:, 32:48]


def _encode(x, w1, b1, w2, b2, wg, blk):
    n = x.shape[0]
    nb = n // blk
    full = lambda *shape: pl.BlockSpec(shape, _im(lambda i: tuple(0 for _ in shape)))
    return pl.pallas_call(
        _enc_body,
        grid=(nb,),
        in_specs=[
            pl.BlockSpec((blk, 128), _im(lambda i: (i, 0))),
            full(32, 128), full(1, 32), full(16, 32), full(1, 16),
            full(16, 48),
        ],
        out_specs=[pl.BlockSpec((blk, F), _im(lambda i: (i, 0)))] * 4,
        out_shape=[jax.ShapeDtypeStruct((n, F), f32)] * 4,
    )(x, w1, b1, w2, b2, wg)


def _comb_body(p0_ref, p1_ref, h_ref, wr_ref, bs_ref, wg_ref,
               h1_ref, g0_ref, g1_ref):
    v = p0_ref[0] + p1_ref[0] + _dot_t(h_ref[...], wr_ref[...]) + bs_ref[...]
    h1 = jnp.maximum(v, 0.0)
    h1_ref[...] = h1
    g = _dot(h1, wg_ref[...])
    g0_ref[...] = g[:, 0:16]
    g1_ref[...] = g[:, 16:32]


def _combine(p, h, wr, bs, wg, blk):
    n = h.shape[0]
    nb = n // blk
    full = lambda *shape: pl.BlockSpec(shape, _im(lambda i: tuple(0 for _ in shape)))
    return pl.pallas_call(
        _comb_body,
        grid=(nb,),
        in_specs=[
            pl.BlockSpec((1, blk, F), _im(lambda i: (0, i, 0))),
            pl.BlockSpec((1, blk, F), _im(lambda i: (1, i, 0))),
            pl.BlockSpec((blk, F), _im(lambda i: (i, 0))),
            full(16, 16), full(1, 16), full(16, 32),
        ],
        out_specs=[pl.BlockSpec((blk, F), _im(lambda i: (i, 0)))] * 3,
        out_shape=[jax.ShapeDtypeStruct((n, F), f32)] * 3,
    )(p, p, h, wr, bs, wg)


def _final_body(p0_ref, p1_ref, h_ref, wr_ref, bs_ref, wp_ref, bp_ref, o_ref):
    v = p0_ref[0] + p1_ref[0] + _dot_t(h_ref[...], wr_ref[...]) + bs_ref[...]
    h1 = jnp.maximum(v, 0.0)
    o_ref[...] = jnp.sum(h1 * wp_ref[...], axis=1, keepdims=True) + bp_ref[...]


def _final(p, h, wr, bs, wp, bp, blk):
    n = h.shape[0]
    nb = n // blk
    full = lambda *shape: pl.BlockSpec(shape, _im(lambda i: tuple(0 for _ in shape)))
    return pl.pallas_call(
        _final_body,
        grid=(nb,),
        in_specs=[
            pl.BlockSpec((1, blk, F), _im(lambda i: (0, i, 0))),
            pl.BlockSpec((1, blk, F), _im(lambda i: (1, i, 0))),
            pl.BlockSpec((blk, F), _im(lambda i: (i, 0))),
            full(16, 16), full(1, 16), full(1, 16), full(1, 1),
        ],
        out_specs=pl.BlockSpec((blk, 1), _im(lambda i: (i, 0))),
        out_shape=jax.ShapeDtypeStruct((n, 1), f32),
    )(p, p, h, wr, bs, wp, bp)


def _make_sc_scatter(rel_dst, nd_list):
    nrel = len(rel_dst)
    ndst = len(nd_list)
    npad_list = [nd + 32 for nd in nd_list]

    scratch = [pltpu.VMEM_SHARED((npad, F), f32) for npad in npad_list]
    scratch += [
        pltpu.VMEM((NCH // 2, CHUNK), jnp.int32),
        pltpu.VMEM((NCH // 2, CHUNK), jnp.int32),
    ]
    scratch += [pltpu.VMEM((CHUNK, F), f32) for _ in range(KBUF)]
    scratch += [pltpu.VMEM((CHUNK, F), f32)]
    scratch += [pltpu.SemaphoreType.DMA for _ in range(KBUF)]
    out_type = [jax.ShapeDtypeStruct((NCORE, nd, F), f32) for nd in nd_list]
    mesh = plsc.VectorSubcoreMesh(core_axis_name="c", subcore_axis_name="s")

    @functools.partial(
        pl.kernel, out_type=out_type, mesh=mesh, scratch_types=scratch,
        compiler_params=pltpu.CompilerParams(use_tc_tiling_on_sc=False))
    def sc_kernel(*refs):
        ins = refs[:3 * nrel]
        outs = refs[3 * nrel:3 * nrel + ndst]
        accs = refs[3 * nrel + ndst:3 * nrel + 2 * ndst]
        rest = refs[3 * nrel + 2 * ndst:]
        srcbuf, dstbuf = rest[0], rest[1]
        rows = rest[2:2 + KBUF]
        zbuf = rest[2 + KBUF]
        sems = rest[3 + KBUF:3 + 2 * KBUF]
        rows0 = rows[0]

        cid = lax.axis_index("c")
        sid = lax.axis_index("s")
        wid = sid * NCORE + cid

        i32 = jnp.int32

        @pl.loop(i32(0), i32(CHUNK))
        def zstore(i):
            zbuf[i, :] = jnp.zeros((F,), f32)

        for acc, npad in zip(accs, npad_list):
            nfull, rem = divmod(npad, CHUNK)

            @pl.loop(sid, i32(nfull), step=i32(NSUB))
            def zcopy(c, acc=acc):
                pltpu.sync_copy(zbuf, acc.at[pl.ds(c * CHUNK, CHUNK)])
            if rem:
                @pl.when(sid == i32(nfull % NSUB))
                def _(acc=acc, off=nfull * CHUNK, rem=rem):
                    pltpu.sync_copy(zbuf.at[pl.ds(i32(0), rem)],
                                    acc.at[pl.ds(i32(off), rem)])

        plsc.subcore_barrier()

        for r in range(nrel):
            tab = ins[3 * r]
            src = ins[3 * r + 1]
            dst = ins[3 * r + 2]
            acc = accs[rel_dst[r]]

            half = NCH // 2
            for h in range(2):
                pltpu.sync_copy(
                    src.at[pl.ds(wid * NCH + h * half, half)], srcbuf)
                pltpu.sync_copy(
                    dst.at[pl.ds(wid * NCH + h * half, half)], dstbuf)

                for b in range(KBUF):
                    pltpu.async_copy(tab.at[srcbuf.at[i32(b)]],
                                     rows[b], sems[b])

                nfull = half // KBUF
                tail = half - nfull * KBUF

                @pl.loop(i32(0), i32(nfull))
                def step(i, tab=tab, acc=acc):
                    j0 = KBUF * i
                    for b in range(KBUF):
                        j = j0 + i32(b)
                        pltpu.make_async_copy(
                            tab.at[srcbuf.at[i32(b)]], rows[b], sems[b]).wait()
                        pltpu.sync_copy(rows[b], acc.at[dstbuf.at[j]],
                                        add=True)
                        pltpu.async_copy(
                            tab.at[srcbuf.at[jnp.minimum(j + KBUF, half - 1)]],
                            rows[b], sems[b])
                for b in range(KBUF):
                    pltpu.make_async_copy(
                        tab.at[srcbuf.at[i32(b)]], rows[b], sems[b]).wait()
                    if b < tail:
                        pltpu.sync_copy(
                            rows[b], acc.at[dstbuf.at[i32(nfull * KBUF + b)]],
                            add=True)

        plsc.subcore_barrier()

        for out, acc, nd in zip(outs, accs, nd_list):
            nfull, rem = divmod(nd, CHUNK)

            @pl.loop(sid, i32(nfull), step=i32(NSUB))
            def wcopy(c, out=out, acc=acc):
                off = pl.multiple_of(c * CHUNK, CHUNK)
                pltpu.sync_copy(acc.at[pl.ds(off, CHUNK)], rows0)
                pltpu.sync_copy(rows0, out.at[cid, pl.ds(off, CHUNK)])
            if rem:
                @pl.when(sid == i32(nfull % NSUB))
                def _(out=out, acc=acc, off=nfull * CHUNK, rem=rem):
                    pltpu.sync_copy(acc.at[pl.ds(i32(off), rem)],
                                    rows0.at[pl.ds(i32(0), rem)])
                    pltpu.sync_copy(rows0.at[pl.ds(i32(0), rem)],
                                    out.at[cid, pl.ds(i32(off), rem)])

    return sc_kernel


def kernel(x_H, x_C, x_Others, encW1, encb1, encW2, encb2, Wrel, brel, Wroot,
           WpH, bpH, WpC, bpC, edge_index_H_H, edge_index_H_C,
           edge_index_H_Others, edge_index_C_H, edge_index_C_C,
           edge_index_C_Others, edge_index_Others_H, edge_index_Others_C,
           edge_index_Others_Others):
    eis = [edge_index_H_H, edge_index_H_C, edge_index_H_Others,
           edge_index_C_H, edge_index_C_C, edge_index_C_Others,
           edge_index_Others_H, edge_index_Others_C, edge_index_Others_Others]
    xs = [x_H, x_C, x_Others]

    pad = E_PAD - E
    srcs, dsts = [], []
    for r in range(9):
        nd = NS_LIST[r % 3]
        ei = eis[r].astype(jnp.int32)
        src = jnp.concatenate([ei[0], jnp.zeros((pad,), jnp.int32)])
        dst = jnp.concatenate([ei[1], jnp.full((pad,), nd, jnp.int32)])
        srcs.append(src.reshape(NTILE * NCH, CHUNK))
        dsts.append(dst.reshape(NTILE * NCH, CHUNK))

    b1 = encb1.reshape(3, 1, 32)
    b2 = encb2.reshape(3, 1, 16)
    wroot_sum = [[Wroot[l, d] + Wroot[l, 3 + d] + Wroot[l, 6 + d]
                  for d in range(3)] for l in range(2)]
    bsum = [[(brel[l, d] + brel[l, 3 + d] + brel[l, 6 + d]).reshape(1, 16)
             for d in range(3)] for l in range(2)]

    h0, g0 = [], []
    for t in range(3):
        wgcat = jnp.concatenate([Wrel[0, 3 * t + k].T for k in range(3)],
                                axis=1)
        res = _encode(xs[t], encW1[t], b1[t], encW2[t], b2[t], wgcat, 4000)
        h0.append(res[0])
        g0.append(res[1:])

    sc0 = _make_sc_scatter(tuple(r % 3 for r in range(9)), NS_LIST)
    args0 = []
    for r in range(9):
        args0 += [g0[r // 3][r % 3], srcs[r], dsts[r]]
    p0 = sc0(*args0)

    h1, g1 = [], []
    for t in range(3):
        wgcat = jnp.concatenate([Wrel[1, 3 * t + k].T for k in range(2)],
                                axis=1)
        res = _combine(p0[t], h0[t], wroot_sum[0][t], bsum[0][t], wgcat, 4000)
        h1.append(res[0])
        g1.append(res[1:])

    rels1 = [3 * s + d for s in range(3) for d in range(2)]
    sc1 = _make_sc_scatter(tuple(r % 3 for r in rels1), (N_H, N_C))
    args1 = []
    for r in rels1:
        args1 += [g1[r // 3][r % 3], srcs[r], dsts[r]]
    p1 = sc1(*args1)

    out_H = _final(p1[0], h1[0], wroot_sum[1][0], bsum[1][0],
                   WpH.reshape(1, 16), bpH.reshape(1, 1), 4000)
    out_C = _final(p1[1], h1[1], wroot_sum[1][1], bsum[1][1],
                   WpC.reshape(1, 16), bpC.reshape(1, 1), 4000)
    return jnp.concatenate([out_H, out_C], axis=0)


# speedup vs baseline: 1.4435x; 1.2394x over previous
import functools

import jax
import jax.numpy as jnp
from jax import lax
from jax.experimental import pallas as pl
from jax.experimental.pallas import tpu as pltpu
from jax.experimental.pallas import tpu_sc as plsc

N_H, N_C, N_O = 40000, 40000, 20000
NS_LIST = (N_H, N_C, N_O)
E = 360000
NCORE, NSUB, NTILE = 2, 16, 32
CHUNK = 128
NCH = 88
E_PAD = NTILE * NCH * CHUNK
KBUF = 8
F = 16
HIGH = None
f32 = jnp.float32


def _i32c(v):
    return jnp.int32(v) if isinstance(v, int) else v


def _im(f):
    return lambda *a: tuple(_i32c(v) for v in f(*a))


def _dot_t(a, w):
    return lax.dot_general(a, w, (((1,), (1,)), ((), ())),
                           preferred_element_type=f32, precision=HIGH)


def _dot(a, b):
    return lax.dot_general(a, b, (((1,), (0,)), ((), ())),
                           preferred_element_type=f32, precision=HIGH)


def _enc_body(x_ref, w1_ref, b1_ref, w2_ref, b2_ref, wg_ref,
              h_ref, g0_ref, g1_ref, g2_ref):
    z = jnp.maximum(_dot_t(x_ref[...], w1_ref[...]) + b1_ref[...], 0.0)
    h = jnp.maximum(_dot_t(z, w2_ref[...]) + b2_ref[...], 0.0)
    h_ref[...] = h
    g = _dot(h, wg_ref[...])
    g0_ref[...] = g[:, 0:16]
    g1_ref[...] = g[:, 16:32]
    g2_ref[...] = g[:, 32:48]


def _encode(x, w1, b1, w2, b2, wg, blk):
    n = x.shape[0]
    nb = n // blk
    full = lambda *shape: pl.BlockSpec(shape, _im(lambda i: tuple(0 for _ in shape)))
    return pl.pallas_call(
        _enc_body,
        grid=(nb,),
        in_specs=[
            pl.BlockSpec((blk, 128), _im(lambda i: (i, 0))),
            full(32, 128), full(1, 32), full(16, 32), full(1, 16),
            full(16, 48),
        ],
        out_specs=[pl.BlockSpec((blk, F), _im(lambda i: (i, 0)))] * 4,
        out_shape=[jax.ShapeDtypeStruct((n, F), f32)] * 4,
    )(x, w1, b1, w2, b2, wg)


def _comb_body(p0_ref, p1_ref, h_ref, wr_ref, bs_ref, wg_ref,
               h1_ref, g0_ref, g1_ref):
    v = p0_ref[0] + p1_ref[0] + _dot_t(h_ref[...], wr_ref[...]) + bs_ref[...]
    h1 = jnp.maximum(v, 0.0)
    h1_ref[...] = h1
    g = _dot(h1, wg_ref[...])
    g0_ref[...] = g[:, 0:16]
    g1_ref[...] = g[:, 16:32]


def _combine(p, h, wr, bs, wg, blk):
    n = h.shape[0]
    nb = n // blk
    full = lambda *shape: pl.BlockSpec(shape, _im(lambda i: tuple(0 for _ in shape)))
    return pl.pallas_call(
        _comb_body,
        grid=(nb,),
        in_specs=[
            pl.BlockSpec((1, blk, F), _im(lambda i: (0, i, 0))),
            pl.BlockSpec((1, blk, F), _im(lambda i: (1, i, 0))),
            pl.BlockSpec((blk, F), _im(lambda i: (i, 0))),
            full(16, 16), full(1, 16), full(16, 32),
        ],
        out_specs=[pl.BlockSpec((blk, F), _im(lambda i: (i, 0)))] * 3,
        out_shape=[jax.ShapeDtypeStruct((n, F), f32)] * 3,
    )(p, p, h, wr, bs, wg)


def _final_body(p0_ref, p1_ref, h_ref, wr_ref, bs_ref, wp_ref, bp_ref, o_ref):
    v = p0_ref[0] + p1_ref[0] + _dot_t(h_ref[...], wr_ref[...]) + bs_ref[...]
    h1 = jnp.maximum(v, 0.0)
    o_ref[...] = jnp.sum(h1 * wp_ref[...], axis=1, keepdims=True) + bp_ref[...]


def _final(p, h, wr, bs, wp, bp, blk):
    n = h.shape[0]
    nb = n // blk
    full = lambda *shape: pl.BlockSpec(shape, _im(lambda i: tuple(0 for _ in shape)))
    return pl.pallas_call(
        _final_body,
        grid=(nb,),
        in_specs=[
            pl.BlockSpec((1, blk, F), _im(lambda i: (0, i, 0))),
            pl.BlockSpec((1, blk, F), _im(lambda i: (1, i, 0))),
            pl.BlockSpec((blk, F), _im(lambda i: (i, 0))),
            full(16, 16), full(1, 16), full(1, 16), full(1, 1),
        ],
        out_specs=pl.BlockSpec((blk, 1), _im(lambda i: (i, 0))),
        out_shape=jax.ShapeDtypeStruct((n, 1), f32),
    )(p, p, h, wr, bs, wp, bp)


def _make_sc_scatter(rel_dst, nd_list):
    nrel = len(rel_dst)
    ndst = len(nd_list)
    npad_list = [nd + 32 for nd in nd_list]

    scratch = [pltpu.VMEM_SHARED((npad, F), f32) for npad in npad_list]
    scratch += [
        pltpu.VMEM((NCH // 2, CHUNK), jnp.int32),
        pltpu.VMEM((NCH // 2, CHUNK), jnp.int32),
    ]
    scratch += [pltpu.VMEM((CHUNK, F), f32) for _ in range(KBUF)]
    scratch += [pltpu.VMEM((CHUNK, F), f32)]
    scratch += [pltpu.SemaphoreType.DMA for _ in range(KBUF)]
    out_type = [jax.ShapeDtypeStruct((NCORE, nd, F), f32) for nd in nd_list]
    mesh = plsc.VectorSubcoreMesh(core_axis_name="c", subcore_axis_name="s")

    @functools.partial(
        pl.kernel, out_type=out_type, mesh=mesh, scratch_types=scratch,
        compiler_params=pltpu.CompilerParams(use_tc_tiling_on_sc=False))
    def sc_kernel(*refs):
        ins = refs[:3 * nrel]
        outs = refs[3 * nrel:3 * nrel + ndst]
        accs = refs[3 * nrel + ndst:3 * nrel + 2 * ndst]
        rest = refs[3 * nrel + 2 * ndst:]
        srcbuf, dstbuf = rest[0], rest[1]
        rows = rest[2:2 + KBUF]
        zbuf = rest[2 + KBUF]
        sems = rest[3 + KBUF:3 + 2 * KBUF]
        rows0 = rows[0]

        cid = lax.axis_index("c")
        sid = lax.axis_index("s")
        wid = sid * NCORE + cid

        i32 = jnp.int32

        @pl.loop(i32(0), i32(CHUNK))
        def zstore(i):
            zbuf[i, :] = jnp.zeros((F,), f32)

        for acc, npad in zip(accs, npad_list):
            nfull, rem = divmod(npad, CHUNK)

            @pl.loop(sid, i32(nfull), step=i32(NSUB))
            def zcopy(c, acc=acc):
                pltpu.sync_copy(zbuf, acc.at[pl.ds(c * CHUNK, CHUNK)])
            if rem:
                @pl.when(sid == i32(nfull % NSUB))
                def _(acc=acc, off=nfull * CHUNK, rem=rem):
                    pltpu.sync_copy(zbuf.at[pl.ds(i32(0), rem)],
                                    acc.at[pl.ds(i32(off), rem)])

        plsc.subcore_barrier()

        for r in range(nrel):
            tab = ins[3 * r]
            src = ins[3 * r + 1]
            dst = ins[3 * r + 2]
            acc = accs[rel_dst[r]]

            half = NCH // 2
            for h in range(2):
                pltpu.sync_copy(
                    src.at[pl.ds(wid * NCH + h * half, half)], srcbuf)
                pltpu.sync_copy(
                    dst.at[pl.ds(wid * NCH + h * half, half)], dstbuf)

                for b in range(KBUF):
                    pltpu.async_copy(tab.at[srcbuf.at[i32(b)]],
                                     rows[b], sems[b])

                nfull = half // KBUF
                tail = half - nfull * KBUF

                @pl.loop(i32(0), i32(nfull))
                def step(i, tab=tab, acc=acc):
                    j0 = KBUF * i
                    for b in range(KBUF):
                        j = j0 + i32(b)
                        pltpu.make_async_copy(
                            tab.at[srcbuf.at[i32(b)]], rows[b], sems[b]).wait()
                        pltpu.sync_copy(rows[b], acc.at[dstbuf.at[j]],
                                        add=True)
                        pltpu.async_copy(
                            tab.at[srcbuf.at[jnp.minimum(j + KBUF, half - 1)]],
                            rows[b], sems[b])
                for b in range(KBUF):
                    pltpu.make_async_copy(
                        tab.at[srcbuf.at[i32(b)]], rows[b], sems[b]).wait()
                    if b < tail:
                        pltpu.sync_copy(
                            rows[b], acc.at[dstbuf.at[i32(nfull * KBUF + b)]],
                            add=True)

        plsc.subcore_barrier()

        for out, acc, nd in zip(outs, accs, nd_list):
            nfull, rem = divmod(nd, CHUNK)

            @pl.loop(sid, i32(nfull), step=i32(NSUB))
            def wcopy(c, out=out, acc=acc):
                off = pl.multiple_of(c * CHUNK, CHUNK)
                pltpu.sync_copy(acc.at[pl.ds(off, CHUNK)], rows0)
                pltpu.sync_copy(rows0, out.at[cid, pl.ds(off, CHUNK)])
            if rem:
                @pl.when(sid == i32(nfull % NSUB))
                def _(out=out, acc=acc, off=nfull * CHUNK, rem=rem):
                    pltpu.sync_copy(acc.at[pl.ds(i32(off), rem)],
                                    rows0.at[pl.ds(i32(0), rem)])
                    pltpu.sync_copy(rows0.at[pl.ds(i32(0), rem)],
                                    out.at[cid, pl.ds(i32(off), rem)])

    return sc_kernel


def kernel(x_H, x_C, x_Others, encW1, encb1, encW2, encb2, Wrel, brel, Wroot,
           WpH, bpH, WpC, bpC, edge_index_H_H, edge_index_H_C,
           edge_index_H_Others, edge_index_C_H, edge_index_C_C,
           edge_index_C_Others, edge_index_Others_H, edge_index_Others_C,
           edge_index_Others_Others):
    eis = [edge_index_H_H, edge_index_H_C, edge_index_H_Others,
           edge_index_C_H, edge_index_C_C, edge_index_C_Others,
           edge_index_Others_H, edge_index_Others_C, edge_index_Others_Others]
    xs = [x_H, x_C, x_Others]

    pad = E_PAD - E
    srcs, dsts = [], []
    for r in range(9):
        nd = NS_LIST[r % 3]
        ei = eis[r].astype(jnp.int32)
        src = jnp.concatenate([ei[0], jnp.zeros((pad,), jnp.int32)])
        dst = jnp.concatenate([ei[1], jnp.full((pad,), nd, jnp.int32)])
        srcs.append(src.reshape(NTILE * NCH, CHUNK))
        dsts.append(dst.reshape(NTILE * NCH, CHUNK))

    b1 = encb1.reshape(3, 1, 32)
    b2 = encb2.reshape(3, 1, 16)
    wroot_sum = [[Wroot[l, d] + Wroot[l, 3 + d] + Wroot[l, 6 + d]
                  for d in range(3)] for l in range(2)]
    bsum = [[(brel[l, d] + brel[l, 3 + d] + brel[l, 6 + d]).reshape(1, 16)
             for d in range(3)] for l in range(2)]

    h0, g0 = [], []
    for t in range(3):
        wgcat = jnp.concatenate([Wrel[0, 3 * t + k].T for k in range(3)],
                                axis=1)
        res = _encode(xs[t], encW1[t], b1[t], encW2[t], b2[t], wgcat, 4000)
        h0.append(res[0])
        g0.append(res[1:])

    p0 = []
    for d in range(3):
        sc = _make_sc_scatter((0, 0, 0), (NS_LIST[d],))
        args = []
        for s in range(3):
            r = 3 * s + d
            args += [g0[s][d], srcs[r], dsts[r]]
        p0.append(sc(*args)[0])

    h1, g1 = [], []
    for t in range(3):
        wgcat = jnp.concatenate([Wrel[1, 3 * t + k].T for k in range(2)],
                                axis=1)
        res = _combine(p0[t], h0[t], wroot_sum[0][t], bsum[0][t], wgcat, 4000)
        h1.append(res[0])
        g1.append(res[1:])

    p1 = []
    for d in range(2):
        sc = _make_sc_scatter((0, 0, 0), (NS_LIST[d],))
        args = []
        for s in range(3):
            r = 3 * s + d
            args += [g1[s][d], srcs[r], dsts[r]]
        p1.append(sc(*args)[0])

    out_H = _final(p1[0], h1[0], wroot_sum[1][0], bsum[1][0],
                   WpH.reshape(1, 16), bpH.reshape(1, 1), 4000)
    out_C = _final(p1[1], h1[1], wroot_sum[1][1], bsum[1][1],
                   WpC.reshape(1, 16), bpC.reshape(1, 1), 4000)
    return jnp.concatenate([out_H, out_C], axis=0)
